# baseline (speedup 1.0000x reference)
import jax
import jax.numpy as jnp
from jax import lax
from jax.experimental import pallas as pl
from jax.experimental.pallas import tpu as pltpu
from jax.experimental.pallas import tpu_sc as plsc

BATCH = 16384
D = 128
NC, NS = 2, 16
NW = NC * NS
CHUNK = 256


def _make_gather4(nrows):
    bpw = nrows // NW
    nchunk = max(1, bpw // CHUNK)
    chunk = bpw // nchunk
    ntask = 4 * nchunk

    def body(uid, iid, mlp_ut, mlp_it, gmf_ut, gmf_it, out,
             idx_u, idx_i, buf0, buf1,
             gsem0, gsem1, wsem0, wsem1):
        wid = lax.axis_index("s") * NC + lax.axis_index("c")
        base = wid * bpw
        pltpu.sync_copy(uid.at[pl.ds(base, bpw)], idx_u)
        pltpu.sync_copy(iid.at[pl.ds(base, bpw)], idx_i)
        bufs = (buf0, buf1)
        gsems = (gsem0, gsem1)
        wsems = (wsem0, wsem1)
        tasks = []
        for ti, (table, idx) in enumerate(((mlp_ut, idx_u), (mlp_it, idx_i),
                                           (gmf_ut, idx_u), (gmf_it, idx_i))):
            for c in range(nchunk):
                tasks.append((table, idx.at[pl.ds(c * chunk, chunk)],
                              out.at[pl.ds(base + c * chunk, chunk),
                                     pl.ds(ti * D, D)]))

        def start_gather(t):
            b = t % 2
            table, idx_sl, _ = tasks[t]
            return pltpu.async_copy(table.at[idx_sl], bufs[b], gsems[b])

        gh = [None] * ntask
        wh = [None] * ntask
        gh[0] = start_gather(0)
        for t in range(ntask):
            b = t % 2
            if t + 1 < ntask:
                if t - 1 >= 0:
                    wh[t - 1].wait()
                gh[t + 1] = start_gather(t + 1)
            gh[t].wait()
            wh[t] = pltpu.async_copy(bufs[b], tasks[t][2], wsems[b])
        if ntask >= 2:
            wh[ntask - 2].wait()
        wh[ntask - 1].wait()

    rows_t = jax.ShapeDtypeStruct((nrows, 4 * D), jnp.float32)
    return pl.kernel(
        body,
        out_type=rows_t,
        mesh=plsc.VectorSubcoreMesh(core_axis_name="c", subcore_axis_name="s"),
        scratch_types=[
            pltpu.VMEM((bpw,), jnp.int32),
            pltpu.VMEM((bpw,), jnp.int32),
            pltpu.VMEM((chunk, D), jnp.float32),
            pltpu.VMEM((chunk, D), jnp.float32),
            pltpu.SemaphoreType.DMA,
            pltpu.SemaphoreType.DMA,
            pltpu.SemaphoreType.DMA,
            pltpu.SemaphoreType.DMA,
        ],
    )


BLK = 1024


def _mlp_body(x, w1, b1, w2, b2, w3, b3, wo, bo, out):
    f32 = jnp.float32
    h = jnp.dot(x[:, 0:2 * D], w1[...], preferred_element_type=f32) + b1[...]
    h = jnp.maximum(h, 0.0)
    h = jnp.maximum(jnp.dot(h, w2[...], preferred_element_type=f32) + b2[...], 0.0)
    h = jnp.maximum(jnp.dot(h, w3[...], preferred_element_type=f32) + b3[...], 0.0)
    g = x[:, 2 * D:3 * D] * x[:, 3 * D:4 * D]
    wog = wo[:, 0:D]
    woh = wo[:, D:D + 32]
    logits = (jnp.sum(g * wog, axis=1) + jnp.sum(h * woh, axis=1) + bo[0, 0])
    out[...] = jax.nn.sigmoid(logits)


def _const(shape):
    return pl.BlockSpec(shape, lambda i: tuple(0 for _ in shape))


def _make_mlp(nrows):
    return pl.pallas_call(
        _mlp_body,
        grid=(nrows // BLK,),
        in_specs=[
            pl.BlockSpec((BLK, 4 * D), lambda i: (i, 0)),
            _const((2 * D, D)), _const((1, D)),
            _const((D, 64)), _const((1, 64)),
            _const((64, 32)), _const((1, 32)),
            _const((1, D + 32)), _const((1, 1)),
        ],
        out_specs=pl.BlockSpec((BLK,), lambda i: (i,)),
        out_shape=jax.ShapeDtypeStruct((nrows,), jnp.float32),
    )


NSPLIT = 2
HALF = BATCH // NSPLIT
_gather_half = _make_gather4(HALF)
_mlp_half = _make_mlp(HALF)


def kernel(user_id, item_id, rating, mlp_user_table, mlp_item_table,
           gmf_user_table, gmf_item_table, W1, b1, W2, b2, W3, b3, Wo, bo):
    b1r = b1.reshape(1, -1)
    b2r = b2.reshape(1, -1)
    b3r = b3.reshape(1, -1)
    wor = Wo.reshape(1, -1)
    bor = bo.reshape(1, 1)
    preds = []
    for s in range(NSPLIT):
        sl = slice(s * HALF, (s + 1) * HALF)
        x = _gather_half(user_id[sl], item_id[sl],
                         mlp_user_table, mlp_item_table,
                         gmf_user_table, gmf_item_table)
        preds.append(_mlp_half(x, W1, b1r, W2, b2r, W3, b3r, wor, bor))
    return (jnp.concatenate(preds), rating)


# speedup vs baseline: 1.0336x; 1.0336x over previous
import jax
import jax.numpy as jnp
from jax import lax
from jax.experimental import pallas as pl
from jax.experimental.pallas import tpu as pltpu
from jax.experimental.pallas import tpu_sc as plsc

BATCH = 16384
D = 128
NC, NS = 2, 16
NW = NC * NS
CHUNK = 256


def _make_gather4(nrows):
    bpw = nrows // NW
    nchunk = max(1, bpw // CHUNK)
    chunk = bpw // nchunk
    ntask = 4 * nchunk

    nbuf = 3

    def body(uid, iid, mlp_ut, mlp_it, gmf_ut, gmf_it, out,
             idx_u, idx_i, buf0, buf1, buf2,
             gsem0, gsem1, gsem2, wsem0, wsem1, wsem2):
        wid = lax.axis_index("s") * NC + lax.axis_index("c")
        base = wid * bpw
        bufs = (buf0, buf1, buf2)
        gsems = (gsem0, gsem1, gsem2)
        wsems = (wsem0, wsem1, wsem2)
        cu = pltpu.async_copy(uid.at[pl.ds(base, bpw)], idx_u, gsems[0])
        ci = pltpu.async_copy(iid.at[pl.ds(base, bpw)], idx_i, gsems[1])
        cu.wait()
        ci.wait()
        tasks = []
        for ti, (table, idx) in enumerate(((mlp_ut, idx_u), (mlp_it, idx_i),
                                           (gmf_ut, idx_u), (gmf_it, idx_i))):
            for c in range(nchunk):
                tasks.append((table, idx.at[pl.ds(c * chunk, chunk)],
                              out.at[pl.ds(base + c * chunk, chunk),
                                     pl.ds(ti * D, D)]))

        def start_gather(t):
            table, idx_sl, _ = tasks[t]
            return pltpu.async_copy(table.at[idx_sl], bufs[t % nbuf],
                                    gsems[t % nbuf])

        gh = [None] * ntask
        wh = [None] * ntask
        for t in range(min(nbuf, ntask)):
            gh[t] = start_gather(t)
        for t in range(ntask):
            gh[t].wait()
            wh[t] = pltpu.async_copy(bufs[t % nbuf], tasks[t][2],
                                     wsems[t % nbuf])
            nt = t + nbuf
            if nt < ntask:
                wh[t].wait()
                gh[nt] = start_gather(nt)
        for t in range(max(0, ntask - nbuf), ntask):
            wh[t].wait()

    rows_t = jax.ShapeDtypeStruct((nrows, 4 * D), jnp.float32)
    return pl.kernel(
        body,
        out_type=rows_t,
        mesh=plsc.VectorSubcoreMesh(core_axis_name="c", subcore_axis_name="s"),
        scratch_types=[
            pltpu.VMEM((bpw,), jnp.int32),
            pltpu.VMEM((bpw,), jnp.int32),
            pltpu.VMEM((chunk, D), jnp.float32),
            pltpu.VMEM((chunk, D), jnp.float32),
            pltpu.VMEM((chunk, D), jnp.float32),
            pltpu.SemaphoreType.DMA,
            pltpu.SemaphoreType.DMA,
            pltpu.SemaphoreType.DMA,
            pltpu.SemaphoreType.DMA,
            pltpu.SemaphoreType.DMA,
            pltpu.SemaphoreType.DMA,
        ],
    )


BLK = 1024


def _mlp_body(x, w1, b1, w2, b2, w3, b3, wo, bo, out):
    f32 = jnp.float32
    h = jnp.dot(x[:, 0:2 * D], w1[...], preferred_element_type=f32) + b1[...]
    h = jnp.maximum(h, 0.0)
    h = jnp.maximum(jnp.dot(h, w2[...], preferred_element_type=f32) + b2[...], 0.0)
    h = jnp.maximum(jnp.dot(h, w3[...], preferred_element_type=f32) + b3[...], 0.0)
    g = x[:, 2 * D:3 * D] * x[:, 3 * D:4 * D]
    wog = wo[:, 0:D]
    woh = wo[:, D:D + 32]
    logits = (jnp.sum(g * wog, axis=1) + jnp.sum(h * woh, axis=1) + bo[0, 0])
    out[...] = jax.nn.sigmoid(logits)


def _const(shape):
    return pl.BlockSpec(shape, lambda i: tuple(0 for _ in shape))


def _make_mlp(nrows):
    return pl.pallas_call(
        _mlp_body,
        grid=(nrows // BLK,),
        in_specs=[
            pl.BlockSpec((BLK, 4 * D), lambda i: (i, 0)),
            _const((2 * D, D)), _const((1, D)),
            _const((D, 64)), _const((1, 64)),
            _const((64, 32)), _const((1, 32)),
            _const((1, D + 32)), _const((1, 1)),
        ],
        out_specs=pl.BlockSpec((BLK,), lambda i: (i,)),
        out_shape=jax.ShapeDtypeStruct((nrows,), jnp.float32),
    )


NSPLIT = 2
HALF = BATCH // NSPLIT
_gather_half = _make_gather4(HALF)
_mlp_half = _make_mlp(HALF)


def kernel(user_id, item_id, rating, mlp_user_table, mlp_item_table,
           gmf_user_table, gmf_item_table, W1, b1, W2, b2, W3, b3, Wo, bo):
    b1r = b1.reshape(1, -1)
    b2r = b2.reshape(1, -1)
    b3r = b3.reshape(1, -1)
    wor = Wo.reshape(1, -1)
    bor = bo.reshape(1, 1)
    preds = []
    for s in range(NSPLIT):
        sl = slice(s * HALF, (s + 1) * HALF)
        x = _gather_half(user_id[sl], item_id[sl],
                         mlp_user_table, mlp_item_table,
                         gmf_user_table, gmf_item_table)
        preds.append(_mlp_half(x, W1, b1r, W2, b2r, W3, b3r, wor, bor))
    return (jnp.concatenate(preds), rating)


# speedup vs baseline: 1.0568x; 1.0224x over previous
import jax
import jax.numpy as jnp
from jax import lax
from jax.experimental import pallas as pl
from jax.experimental.pallas import tpu as pltpu
from jax.experimental.pallas import tpu_sc as plsc

BATCH = 16384
D = 128
NC, NS = 2, 16
NW = NC * NS
CHUNK = 256


def _make_gather4(nrows):
    bpw = nrows // NW
    nchunk = max(1, bpw // CHUNK)
    chunk = bpw // nchunk
    ntask = 4 * nchunk

    nbuf = 3

    def body(uid, iid, mlp_ut, mlp_it, gmf_ut, gmf_it, out,
             idx_u, idx_i, buf0, buf1, buf2,
             gsem0, gsem1, gsem2, wsem0, wsem1, wsem2):
        wid = lax.axis_index("s") * NC + lax.axis_index("c")
        base = wid * bpw
        bufs = (buf0, buf1, buf2)
        gsems = (gsem0, gsem1, gsem2)
        wsems = (wsem0, wsem1, wsem2)
        cu = pltpu.async_copy(uid.at[pl.ds(base, bpw)], idx_u, gsems[0])
        ci = pltpu.async_copy(iid.at[pl.ds(base, bpw)], idx_i, gsems[1])
        cu.wait()
        ci.wait()
        tasks = []
        for ti, (table, idx) in enumerate(((mlp_ut, idx_u), (mlp_it, idx_i),
                                           (gmf_ut, idx_u), (gmf_it, idx_i))):
            for c in range(nchunk):
                tasks.append((table, idx.at[pl.ds(c * chunk, chunk)],
                              out.at[pl.ds(base + c * chunk, chunk),
                                     pl.ds(ti * D, D)]))

        def start_gather(t):
            table, idx_sl, _ = tasks[t]
            return pltpu.async_copy(table.at[idx_sl], bufs[t % nbuf],
                                    gsems[t % nbuf])

        gh = [None] * ntask
        wh = [None] * ntask
        for t in range(min(nbuf, ntask)):
            gh[t] = start_gather(t)
        for t in range(ntask):
            gh[t].wait()
            wh[t] = pltpu.async_copy(bufs[t % nbuf], tasks[t][2],
                                     wsems[t % nbuf])
            nt = t + nbuf
            if nt < ntask:
                wh[t].wait()
                gh[nt] = start_gather(nt)
        for t in range(max(0, ntask - nbuf), ntask):
            wh[t].wait()

    rows_t = jax.ShapeDtypeStruct((nrows, 4 * D), jnp.float32)
    return pl.kernel(
        body,
        out_type=rows_t,
        mesh=plsc.VectorSubcoreMesh(core_axis_name="c", subcore_axis_name="s"),
        scratch_types=[
            pltpu.VMEM((bpw,), jnp.int32),
            pltpu.VMEM((bpw,), jnp.int32),
            pltpu.VMEM((chunk, D), jnp.float32),
            pltpu.VMEM((chunk, D), jnp.float32),
            pltpu.VMEM((chunk, D), jnp.float32),
            pltpu.SemaphoreType.DMA,
            pltpu.SemaphoreType.DMA,
            pltpu.SemaphoreType.DMA,
            pltpu.SemaphoreType.DMA,
            pltpu.SemaphoreType.DMA,
            pltpu.SemaphoreType.DMA,
        ],
    )


BLK = 1024


def _mlp_body(x, w1, b1, w2, b2, w3, b3, wo, bo, prev, out):
    f32 = jnp.float32
    h = jnp.dot(x[:, 0:2 * D], w1[...], preferred_element_type=f32) + b1[...]
    h = jnp.maximum(h, 0.0)
    h = jnp.maximum(jnp.dot(h, w2[...], preferred_element_type=f32) + b2[...], 0.0)
    h = jnp.maximum(jnp.dot(h, w3[...], preferred_element_type=f32) + b3[...], 0.0)
    g = x[:, 2 * D:3 * D] * x[:, 3 * D:4 * D]
    logits = (jnp.dot(g, wo[0:D, :], preferred_element_type=f32)
              + jnp.dot(h, wo[D:D + 32, :], preferred_element_type=f32)
              + bo[0])
    out[...] = jax.nn.sigmoid(logits[:, 0])


def _const(shape):
    return pl.BlockSpec(shape, lambda i: tuple(0 for _ in shape))


def _make_mlp(nrows, blk_off):
    return pl.pallas_call(
        _mlp_body,
        grid=(nrows // BLK,),
        in_specs=[
            pl.BlockSpec((BLK, 4 * D), lambda i: (i, 0)),
            _const((2 * D, D)), _const((D,)),
            _const((D, 64)), _const((64,)),
            _const((64, 32)), _const((32,)),
            _const((D + 32, 1)),
            pl.BlockSpec(memory_space=pltpu.SMEM),
            pl.BlockSpec(memory_space=pl.ANY),
        ],
        out_specs=pl.BlockSpec((BLK,), lambda i, o=blk_off: (i + o,)),
        out_shape=jax.ShapeDtypeStruct((BATCH,), jnp.float32),
        input_output_aliases={9: 0},
    )


NSPLIT = 2
HALF = BATCH // NSPLIT
_gather_half = _make_gather4(HALF)
_mlp_halves = [_make_mlp(HALF, s * (HALF // BLK)) for s in range(NSPLIT)]


def kernel(user_id, item_id, rating, mlp_user_table, mlp_item_table,
           gmf_user_table, gmf_item_table, W1, b1, W2, b2, W3, b3, Wo, bo):
    pred = jnp.zeros((BATCH,), jnp.float32)
    for s in range(NSPLIT):
        sl = slice(s * HALF, (s + 1) * HALF)
        x = _gather_half(user_id[sl], item_id[sl],
                         mlp_user_table, mlp_item_table,
                         gmf_user_table, gmf_item_table)
        pred = _mlp_halves[s](x, W1, b1, W2, b2, W3, b3, Wo, bo, pred)
    return (pred, rating)


# speedup vs baseline: 1.0754x; 1.0176x over previous
import jax
import jax.numpy as jnp
from jax import lax
from jax.experimental import pallas as pl
from jax.experimental.pallas import tpu as pltpu
from jax.experimental.pallas import tpu_sc as plsc

BATCH = 16384
D = 128
NC, NS = 2, 16
NW = NC * NS
CHUNK = 256


def _make_gather4(nrows, row_off):
    bpw = nrows // NW
    nchunk = max(1, bpw // CHUNK)
    chunk = bpw // nchunk
    ntask = 4 * nchunk

    nbuf = 3

    def body(uid, iid, mlp_ut, mlp_it, gmf_ut, gmf_it, out,
             idx_u, idx_i, buf0, buf1, buf2,
             gsem0, gsem1, gsem2, wsem0, wsem1, wsem2):
        wid = lax.axis_index("s") * NC + lax.axis_index("c")
        base = wid * bpw
        bufs = (buf0, buf1, buf2)
        gsems = (gsem0, gsem1, gsem2)
        wsems = (wsem0, wsem1, wsem2)
        cu = pltpu.async_copy(uid.at[pl.ds(row_off + base, bpw)], idx_u, gsems[0])
        ci = pltpu.async_copy(iid.at[pl.ds(row_off + base, bpw)], idx_i, gsems[1])
        cu.wait()
        ci.wait()
        tasks = []
        for ti, (table, idx) in enumerate(((mlp_ut, idx_u), (mlp_it, idx_i),
                                           (gmf_ut, idx_u), (gmf_it, idx_i))):
            for c in range(nchunk):
                tasks.append((table, idx.at[pl.ds(c * chunk, chunk)],
                              out.at[pl.ds(base + c * chunk, chunk),
                                     pl.ds(ti * D, D)]))

        def start_gather(t):
            table, idx_sl, _ = tasks[t]
            return pltpu.async_copy(table.at[idx_sl], bufs[t % nbuf],
                                    gsems[t % nbuf])

        gh = [None] * ntask
        wh = [None] * ntask
        for t in range(min(nbuf, ntask)):
            gh[t] = start_gather(t)
        for t in range(ntask):
            gh[t].wait()
            wh[t] = pltpu.async_copy(bufs[t % nbuf], tasks[t][2],
                                     wsems[t % nbuf])
            nt = t + nbuf
            if nt < ntask:
                wh[t].wait()
                gh[nt] = start_gather(nt)
        for t in range(max(0, ntask - nbuf), ntask):
            wh[t].wait()

    rows_t = jax.ShapeDtypeStruct((nrows, 4 * D), jnp.float32)
    return pl.kernel(
        body,
        out_type=rows_t,
        mesh=plsc.VectorSubcoreMesh(core_axis_name="c", subcore_axis_name="s"),
        scratch_types=[
            pltpu.VMEM((bpw,), jnp.int32),
            pltpu.VMEM((bpw,), jnp.int32),
            pltpu.VMEM((chunk, D), jnp.float32),
            pltpu.VMEM((chunk, D), jnp.float32),
            pltpu.VMEM((chunk, D), jnp.float32),
            pltpu.SemaphoreType.DMA,
            pltpu.SemaphoreType.DMA,
            pltpu.SemaphoreType.DMA,
            pltpu.SemaphoreType.DMA,
            pltpu.SemaphoreType.DMA,
            pltpu.SemaphoreType.DMA,
        ],
    )


BLK = 1024


def _mlp_compute(x, w1, b1, w2, b2, w3, b3, wo, bo, out):
    f32 = jnp.float32
    h = jnp.dot(x[:, 0:2 * D], w1[...], preferred_element_type=f32) + b1[...]
    h = jnp.maximum(h, 0.0)
    h = jnp.maximum(jnp.dot(h, w2[...], preferred_element_type=f32) + b2[...], 0.0)
    h = jnp.maximum(jnp.dot(h, w3[...], preferred_element_type=f32) + b3[...], 0.0)
    g = x[:, 2 * D:3 * D] * x[:, 3 * D:4 * D]
    logits = (jnp.dot(g, wo[0:D, :], preferred_element_type=f32)
              + jnp.dot(h, wo[D:D + 32, :], preferred_element_type=f32)
              + bo[0])
    out[...] = jax.nn.sigmoid(logits[:, 0])


def _mlp_body_first(x, w1, b1, w2, b2, w3, b3, wo, bo, out):
    _mlp_compute(x, w1, b1, w2, b2, w3, b3, wo, bo, out)


def _mlp_body(x, w1, b1, w2, b2, w3, b3, wo, bo, prev, out):
    _mlp_compute(x, w1, b1, w2, b2, w3, b3, wo, bo, out)


def _const(shape):
    return pl.BlockSpec(shape, lambda i: tuple(0 for _ in shape))


def _make_mlp(nrows, blk_off, aliased):
    specs = [
        pl.BlockSpec((BLK, 4 * D), lambda i: (i, 0)),
        _const((2 * D, D)), _const((D,)),
        _const((D, 64)), _const((64,)),
        _const((64, 32)), _const((32,)),
        _const((D + 32, 1)),
        pl.BlockSpec(memory_space=pltpu.SMEM),
    ]
    if aliased:
        specs.append(pl.BlockSpec(memory_space=pl.ANY))
    return pl.pallas_call(
        _mlp_body if aliased else _mlp_body_first,
        grid=(nrows // BLK,),
        in_specs=specs,
        out_specs=pl.BlockSpec((BLK,), lambda i, o=blk_off: (i + o,)),
        out_shape=jax.ShapeDtypeStruct((BATCH,), jnp.float32),
        input_output_aliases={9: 0} if aliased else {},
    )


NSPLIT = 2
HALF = BATCH // NSPLIT
_gather_halves = [_make_gather4(HALF, s * HALF) for s in range(NSPLIT)]
_mlp_halves = [_make_mlp(HALF, s * (HALF // BLK), s > 0) for s in range(NSPLIT)]


def kernel(user_id, item_id, rating, mlp_user_table, mlp_item_table,
           gmf_user_table, gmf_item_table, W1, b1, W2, b2, W3, b3, Wo, bo):
    pred = None
    for s in range(NSPLIT):
        x = _gather_halves[s](user_id, item_id,
                              mlp_user_table, mlp_item_table,
                              gmf_user_table, gmf_item_table)
        args = (x, W1, b1, W2, b2, W3, b3, Wo, bo)
        pred = _mlp_halves[s](*args) if s == 0 else _mlp_halves[s](*args, pred)
    return (pred, rating)
